# Initial kernel scaffold; baseline (speedup 1.0000x reference)
import jax
import jax.numpy as jnp
from jax.experimental import pallas as pl


def kernel(input_token, input_obj_id, segment_label, token_type, word_emb, obj_emb, rel_emb, type_emb):
    raise NotImplementedError("write your pallas kernel here")


# speedup vs baseline: 1.2140x; 1.2140x over previous
import functools

import jax
import jax.numpy as jnp
from jax import lax
from jax.experimental import pallas as pl
from jax.experimental.pallas import tpu as pltpu
from jax.experimental.pallas import tpu_sc as plsc

B, L, H = 4096, 50, 128
NC, NS = 2, 16
NW = NC * NS
TOK = B * L
TPW = TOK // NW
CH = 128
NCHUNK = TPW // CH


def _sc_body(it_hbm, io_hbm, ir_hbm, ix_hbm, wtab, otab, rtab, xtab,
             emb_out, ie_out,
             it_v, io_v, ir_v, ix_v, wb, rb, xb, acc, gsem, wsem):
    c = lax.axis_index("c")
    s = lax.axis_index("s")
    wid = s * NC + c

    pltpu.sync_copy(it_hbm.at[wid], it_v)
    pltpu.sync_copy(io_hbm.at[wid], io_v)
    pltpu.sync_copy(ir_hbm.at[wid], ir_v)
    pltpu.sync_copy(ix_hbm.at[wid], ix_v)

    def chunk(j, carry):
        base = wid * TPW + j * CH
        cw = pltpu.async_copy(wtab.at[it_v.at[j]], wb, gsem)
        co = pltpu.async_copy(otab.at[io_v.at[j]], acc, gsem)
        cr = pltpu.async_copy(rtab.at[ir_v.at[j]], rb, gsem)
        cx = pltpu.async_copy(xtab.at[ix_v.at[j]], xb, gsem)
        cw.wait()
        co.wait()
        cr.wait()
        cx.wait()
        w1 = pltpu.async_copy(wb, ie_out.at[pl.ds(base, CH)], wsem)

        def row(r, carry2):
            for cc in range(H // 16):
                sl = pl.ds(cc * 16, 16)
                plsc.addupdate(acc.at[r, sl], wb[r, sl] + rb[r, sl] + xb[r, sl])
            return carry2

        lax.fori_loop(0, CH, row, 0)
        w2 = pltpu.async_copy(acc, emb_out.at[pl.ds(base, CH)], wsem)
        w1.wait()
        w2.wait()
        return carry

    lax.fori_loop(0, NCHUNK, chunk, 0)


@functools.partial(
    pl.kernel,
    out_type=(jax.ShapeDtypeStruct((TOK, H), jnp.float32),
              jax.ShapeDtypeStruct((TOK, H), jnp.float32)),
    mesh=plsc.VectorSubcoreMesh(core_axis_name="c", subcore_axis_name="s"),
    scratch_types=(
        pltpu.VMEM((NCHUNK, CH), jnp.int32),
        pltpu.VMEM((NCHUNK, CH), jnp.int32),
        pltpu.VMEM((NCHUNK, CH), jnp.int32),
        pltpu.VMEM((NCHUNK, CH), jnp.int32),
        pltpu.VMEM((CH, H), jnp.float32),
        pltpu.VMEM((CH, H), jnp.float32),
        pltpu.VMEM((CH, H), jnp.float32),
        pltpu.VMEM((CH, H), jnp.float32),
        pltpu.SemaphoreType.DMA,
        pltpu.SemaphoreType.DMA,
    ),
)
def _sc_embed(*args):
    _sc_body(*args)


def kernel(input_token, input_obj_id, segment_label, token_type,
           word_emb, obj_emb, rel_emb, type_emb):
    it = input_token.reshape(NW, NCHUNK, CH).astype(jnp.int32)
    io = input_obj_id.reshape(NW, NCHUNK, CH).astype(jnp.int32)
    ir = segment_label.reshape(NW, NCHUNK, CH).astype(jnp.int32)
    ix = token_type.reshape(NW, NCHUNK, CH).astype(jnp.int32)
    emb, ie = _sc_embed(it, io, ir, ix, word_emb, obj_emb, rel_emb, type_emb)
    return emb.reshape(B, L, H), ie.reshape(B, L, H)


# speedup vs baseline: 4.3392x; 3.5742x over previous
import functools

import jax
import jax.numpy as jnp
from jax import lax
from jax.experimental import pallas as pl
from jax.experimental.pallas import tpu as pltpu
from jax.experimental.pallas import tpu_sc as plsc

B, L, H = 4096, 50, 128
NC, NS = 2, 16
NW = NC * NS
TOK = B * L
TPW = TOK // NW
CH = 128
NCHUNK = TPW // CH
NREL, NTYP = 17, 4
NRT = NREL * NTYP


def _sc_body(it_hbm, io_hbm, irt_hbm, wtab, otab, rtab, xtab,
             emb_out, ie_out,
             itv, iov, irtv, wb0, wb1, ac0, ac1, rtb0, rtb1,
             rt_hbm, g0, g1, w0, w1):
    c = lax.axis_index("c")
    s = lax.axis_index("s")
    wid = s * NC + c

    @pl.when(s == 0)
    def _():
        pltpu.sync_copy(rtab, wb0.at[pl.ds(0, NREL)])
        pltpu.sync_copy(xtab, wb0.at[pl.ds(NREL, NTYP)])

        def fuse(r, carry):
            for t in range(NTYP):
                for cc in range(H // 16):
                    sl = pl.ds(cc * 16, 16)
                    ac0[r * NTYP + t, sl] = wb0[r, sl] + wb0[NREL + t, sl]
            return carry

        lax.fori_loop(0, NREL, fuse, 0)
        pltpu.sync_copy(ac0.at[pl.ds(0, NRT)], rt_hbm)

    pltpu.sync_copy(it_hbm.at[wid], itv)
    pltpu.sync_copy(io_hbm.at[wid], iov)
    pltpu.sync_copy(irt_hbm.at[wid], irtv)
    plsc.subcore_barrier()

    wbs, accs, rtbs = (wb0, wb1), (ac0, ac1), (rtb0, rtb1)
    gs, ws = (g0, g1), (w0, w1)

    def fire_g(j, b):
        pltpu.async_copy(wtab.at[itv.at[j]], wbs[b], gs[b])
        pltpu.async_copy(otab.at[iov.at[j]], accs[b], gs[b])
        pltpu.async_copy(rt_hbm.at[irtv.at[j]], rtbs[b], gs[b])

    def wait_g(j, b):
        pltpu.make_async_copy(wtab.at[itv.at[j]], wbs[b], gs[b]).wait()
        pltpu.make_async_copy(otab.at[iov.at[j]], accs[b], gs[b]).wait()
        pltpu.make_async_copy(rt_hbm.at[irtv.at[j]], rtbs[b], gs[b]).wait()

    def fire_w(j, b):
        base = wid * TPW + j * CH
        pltpu.async_copy(wbs[b], ie_out.at[pl.ds(base, CH)], ws[b])
        pltpu.async_copy(accs[b], emb_out.at[pl.ds(base, CH)], ws[b])

    def wait_w(j, b):
        base = wid * TPW + j * CH
        pltpu.make_async_copy(wbs[b], ie_out.at[pl.ds(base, CH)], ws[b]).wait()
        pltpu.make_async_copy(accs[b], emb_out.at[pl.ds(base, CH)], ws[b]).wait()

    def compute(j, b):
        wb, acc, rtb = wbs[b], accs[b], rtbs[b]

        @plsc.parallel_loop(0, CH, unroll=2)
        def _pass(t):
            for cc in range(H // 16):
                sl = pl.ds(cc * 16, 16)
                plsc.addupdate(acc.at[t, sl], wb[t, sl] + rtb[t, sl])

    fire_g(0, 0)

    def pair(p, carry):
        for b in range(2):
            j = 2 * p + b
            nb = (b + 1) % 2

            @pl.when(j + 1 < NCHUNK)
            def _():
                @pl.when(j >= 1)
                def _():
                    wait_w(j - 1, nb)

                fire_g(j + 1, nb)

            wait_g(j, b)
            compute(j, b)
            fire_w(j, b)
        return carry

    lax.fori_loop(0, NCHUNK // 2, pair, 0)
    wait_w(NCHUNK - 2, 0)
    wait_w(NCHUNK - 1, 1)


@functools.partial(
    pl.kernel,
    out_type=(jax.ShapeDtypeStruct((TOK, H), jnp.float32),
              jax.ShapeDtypeStruct((TOK, H), jnp.float32)),
    mesh=plsc.VectorSubcoreMesh(core_axis_name="c", subcore_axis_name="s"),
    scratch_types=(
        pltpu.VMEM((NCHUNK, CH), jnp.int32),
        pltpu.VMEM((NCHUNK, CH), jnp.int32),
        pltpu.VMEM((NCHUNK, CH), jnp.int32),
        pltpu.VMEM((CH, H), jnp.float32),
        pltpu.VMEM((CH, H), jnp.float32),
        pltpu.VMEM((CH, H), jnp.float32),
        pltpu.VMEM((CH, H), jnp.float32),
        pltpu.VMEM((CH, H), jnp.float32),
        pltpu.VMEM((CH, H), jnp.float32),
        pltpu.HBM((NRT, H), jnp.float32),
        pltpu.SemaphoreType.DMA,
        pltpu.SemaphoreType.DMA,
        pltpu.SemaphoreType.DMA,
        pltpu.SemaphoreType.DMA,
    ),
)
def _sc_embed(*args):
    _sc_body(*args)


def kernel(input_token, input_obj_id, segment_label, token_type,
           word_emb, obj_emb, rel_emb, type_emb):
    it = input_token.reshape(NW, NCHUNK, CH).astype(jnp.int32)
    io = input_obj_id.reshape(NW, NCHUNK, CH).astype(jnp.int32)
    irt = (segment_label.astype(jnp.int32) * NTYP
           + token_type.astype(jnp.int32)).reshape(NW, NCHUNK, CH)
    emb, ie = _sc_embed(it, io, irt, word_emb, obj_emb, rel_emb, type_emb)
    return emb.reshape(B, L, H), ie.reshape(B, L, H)


# speedup vs baseline: 6.2250x; 1.4346x over previous
import functools

import jax
import jax.numpy as jnp
from jax import lax
from jax.experimental import pallas as pl
from jax.experimental.pallas import tpu as pltpu
from jax.experimental.pallas import tpu_sc as plsc

B, L, H = 4096, 50, 128
NC, NS = 2, 16
NW = NC * NS
TOK = B * L
TPW = TOK // NW
CH = 128
NCHUNK = TPW // CH
NREL, NTYP = 17, 4
NRT = NREL * NTYP


def _sc_body(it_hbm, io_hbm, irt_hbm, wtab, otab, rtab, xtab,
             emb_out, ie_out,
             itv, iov, irtv, wb0, wb1, ac0, ac1, rtb0, rtb1,
             rt_hbm, g0, g1, w0, w1):
    c = lax.axis_index("c")
    s = lax.axis_index("s")
    wid = s * NC + c

    @pl.when(s == 0)
    def _():
        pltpu.sync_copy(rtab, wb0.at[pl.ds(0, NREL)])
        pltpu.sync_copy(xtab, wb0.at[pl.ds(NREL, NTYP)])

        def fuse(r, carry):
            for t in range(NTYP):
                for cc in range(H // 16):
                    sl = pl.ds(cc * 16, 16)
                    ac0[r * NTYP + t, sl] = wb0[r, sl] + wb0[NREL + t, sl]
            return carry

        lax.fori_loop(0, NREL, fuse, 0)
        pltpu.sync_copy(ac0.at[pl.ds(0, NRT)], rt_hbm)

    pltpu.sync_copy(it_hbm.at[wid], itv)
    pltpu.sync_copy(io_hbm.at[wid], iov)
    pltpu.sync_copy(irt_hbm.at[wid], irtv)
    plsc.subcore_barrier()

    wbs, accs, rtbs = (wb0, wb1), (ac0, ac1), (rtb0, rtb1)
    gs, ws = (g0, g1), (w0, w1)

    def fire_g(j, b):
        pltpu.async_copy(wtab.at[itv.at[j]], wbs[b], gs[b])
        pltpu.async_copy(otab.at[iov.at[j]], accs[b], gs[b])
        pltpu.async_copy(rt_hbm.at[irtv.at[j]], rtbs[b], gs[b])

    def wait_g(j, b):
        pltpu.make_async_copy(wtab.at[itv.at[j]], wbs[b], gs[b]).wait()
        pltpu.make_async_copy(otab.at[iov.at[j]], accs[b], gs[b]).wait()
        pltpu.make_async_copy(rt_hbm.at[irtv.at[j]], rtbs[b], gs[b]).wait()

    def fire_w(j, b):
        base = wid * TPW + j * CH
        pltpu.async_copy(wbs[b], ie_out.at[pl.ds(base, CH)], ws[b])
        pltpu.async_copy(accs[b], emb_out.at[pl.ds(base, CH)], ws[b])

    def wait_w(j, b):
        base = wid * TPW + j * CH
        pltpu.make_async_copy(wbs[b], ie_out.at[pl.ds(base, CH)], ws[b]).wait()
        pltpu.make_async_copy(accs[b], emb_out.at[pl.ds(base, CH)], ws[b]).wait()

    def compute(j, b):
        wb, acc, rtb = wbs[b], accs[b], rtbs[b]

        @plsc.parallel_loop(0, CH, unroll=2)
        def _pass(t):
            for cc in range(H // 16):
                sl = pl.ds(cc * 16, 16)
                plsc.addupdate(acc.at[t, sl], wb[t, sl] + rtb[t, sl])

    fire_g(0, 0)

    def pair(p, carry):
        for b in range(2):
            j = 2 * p + b
            nb = (b + 1) % 2

            @pl.when(j + 1 < NCHUNK)
            def _():
                @pl.when(j >= 1)
                def _():
                    wait_w(j - 1, nb)

                fire_g(j + 1, nb)

            wait_g(j, b)
            compute(j, b)
            fire_w(j, b)
        return carry

    lax.fori_loop(0, NCHUNK // 2, pair, 0)
    wait_w(NCHUNK - 2, 0)
    wait_w(NCHUNK - 1, 1)


@functools.partial(
    pl.kernel,
    out_type=(jax.ShapeDtypeStruct((TOK, H), jnp.float32),
              jax.ShapeDtypeStruct((TOK, H), jnp.float32)),
    mesh=plsc.VectorSubcoreMesh(core_axis_name="c", subcore_axis_name="s"),
    scratch_types=(
        pltpu.VMEM((NCHUNK, CH), jnp.int32),
        pltpu.VMEM((NCHUNK, CH), jnp.int32),
        pltpu.VMEM((NCHUNK, CH), jnp.int32),
        pltpu.VMEM((CH, H), jnp.float32),
        pltpu.VMEM((CH, H), jnp.float32),
        pltpu.VMEM((CH, H), jnp.float32),
        pltpu.VMEM((CH, H), jnp.float32),
        pltpu.VMEM((CH, H), jnp.float32),
        pltpu.VMEM((CH, H), jnp.float32),
        pltpu.HBM((NRT, H), jnp.float32),
        pltpu.SemaphoreType.DMA,
        pltpu.SemaphoreType.DMA,
        pltpu.SemaphoreType.DMA,
        pltpu.SemaphoreType.DMA,
    ),
)
def _sc_embed(*args):
    _sc_body(*args)


def kernel(input_token, input_obj_id, segment_label, token_type,
           word_emb, obj_emb, rel_emb, type_emb):
    it = input_token.T.reshape(NW, NCHUNK, CH).astype(jnp.int32)
    io = input_obj_id.T.reshape(NW, NCHUNK, CH).astype(jnp.int32)
    irt = (segment_label.astype(jnp.int32) * NTYP
           + token_type.astype(jnp.int32)).T.reshape(NW, NCHUNK, CH)
    emb, ie = _sc_embed(it, io, irt, word_emb, obj_emb, rel_emb, type_emb)
    emb = emb.reshape(L, B, H).transpose(1, 0, 2)
    ie = ie.reshape(L, B, H).transpose(1, 0, 2)
    return emb, ie
